# Initial kernel scaffold; baseline (speedup 1.0000x reference)
import jax
import jax.numpy as jnp
from jax.experimental import pallas as pl


def kernel(x1, x2, x3, W0, b0, W1, b1, W2, b2):
    raise NotImplementedError("write your pallas kernel here")


# speedup vs baseline: 10.5989x; 10.5989x over previous
import functools

import numpy as np
import jax
import jax.numpy as jnp
from jax.experimental import pallas as pl
from jax.experimental.pallas import tpu as pltpu

_F = 8
_DEC_LO = np.array([-0.010597401784997278, 0.032883011666982945, 0.030841381835986965, -0.18703481171888114, -0.02798376941698385, 0.6308807679295904, 0.7148465705525415, 0.23037781330885523], dtype=np.float64)
_REC_LO = _DEC_LO[::-1].copy()
_SIGN = np.array([(-1.0) ** (k + 1) for k in range(_F)], dtype=np.float64)
_DEC_HI = _REC_LO * _SIGN
_REC_HI = _DEC_HI[::-1].copy()


def _conv_valid(x, f):
    n = x.shape[1]
    m = n - _F + 1
    out = np.zeros((x.shape[0], m), dtype=x.dtype)
    for i in range(_F):
        out += f[i] * x[:, _F - 1 - i:_F - 1 - i + m]
    return out


def _conv_full(x, f):
    return _conv_valid(np.pad(x, ((0, 0), (_F - 1, _F - 1))), f)


def _dwt_np(x):
    ext = np.pad(x, ((0, 0), (_F - 1, _F - 1)), mode='symmetric')
    return _conv_valid(ext, _DEC_LO)[:, 1::2], _conv_valid(ext, _DEC_HI)[:, 1::2]


def _idwt_np(cA, cD):
    B, n = cA.shape
    ua = np.zeros((B, 2 * n), cA.dtype)
    ua[:, ::2] = cA
    ud = np.zeros((B, 2 * n), cD.dtype)
    ud[:, ::2] = cD
    full = _conv_full(ua, _REC_LO) + _conv_full(ud, _REC_HI)
    return full[:, _F - 2:2 * n]


@functools.lru_cache(maxsize=1)
def _build_stage_matrices(L=2048, max_level=8):
    ns = []
    n = L
    for _ in range(max_level):
        n = (n + _F - 1) // 2
        ns.append(n)
    casc = []
    prev = L
    for n in ns:
        eye = np.eye(prev)
        a, d = _dwt_np(eye)
        casc.append(np.concatenate([a, d], axis=1).astype(np.float32))
        prev = n
    lmat, hmat = {}, {}
    for m in set(ns):
        eye = np.eye(m)
        z = np.zeros((m, m))
        lmat[m] = _idwt_np(eye, z).astype(np.float32)
        hmat[m] = _idwt_np(z, eye).astype(np.float32)
    return tuple(ns), tuple(casc), lmat, hmat


def _tc_body(ns, x1_ref, *refs):
    max_level = len(ns)
    casc_refs = refs[:max_level]
    ms = sorted(set(ns))
    l_refs = dict(zip(ms, refs[max_level:max_level + len(ms)]))
    h_refs = dict(zip(ms, refs[max_level + len(ms):max_level + 2 * len(ms)]))
    (w0t_ref, b0_ref, w1t_ref, b1_ref, w2t_ref, b2_ref,
     out_ref) = refs[max_level + 2 * len(ms):]

    def dot16(u, m_ref):
        return jnp.dot(u.astype(jnp.bfloat16), m_ref[...],
                       preferred_element_type=jnp.float32)

    a = x1_ref[...]
    cds = []
    recs = []
    for layer in range(1, max_level + 1):
        n = ns[layer - 1]
        both = dot16(a, casc_refs[layer - 1])
        a_new = both[:, :n]
        d_new = both[:, n:]
        if layer >= 2:
            r = dot16(d_new, h_refs[n])
            chain = [(ns[layer - 2], cds[-1])]
            chain += [(ns[j], None) for j in range(layer - 3, -1, -1)]
            for m, d in chain:
                if r.shape[1] == m + 1:
                    r = r[:, :m]
                rr = dot16(r, l_refs[m])
                if d is not None:
                    rr = rr + dot16(d, h_refs[m])
                r = rr
            recs.append(r)
        cds.append(d_new)
        a = a_new
    h = jnp.concatenate(recs, axis=1)
    h0 = dot16(jnp.maximum(h, 0.0), w0t_ref) + b0_ref[...]
    h1 = dot16(jnp.maximum(h0, 0.0), w1t_ref) + b1_ref[...]
    out_ref[...] = dot16(jnp.maximum(h1, 0.0), w2t_ref) + b2_ref[...]


def kernel(x1, x2, x3, W0, b0, W1, b1, W2, b2):
    del x2, x3
    ns, casc, lmat, hmat = _build_stage_matrices()
    B = x1.shape[0]
    bf = jnp.bfloat16
    ms = sorted(set(ns))
    consts = ([jnp.asarray(c).astype(bf) for c in casc]
              + [jnp.asarray(lmat[m]).astype(bf) for m in ms]
              + [jnp.asarray(hmat[m]).astype(bf) for m in ms])
    args = ([x1] + consts
            + [W0.T.astype(bf), b0.reshape(1, -1),
               W1.T.astype(bf), b1.reshape(1, -1),
               W2.T.astype(bf), b2.reshape(1, -1)])
    body = functools.partial(_tc_body, ns)
    return pl.pallas_call(
        body,
        out_shape=jax.ShapeDtypeStruct((B, W2.shape[0]), jnp.float32),
        compiler_params=pltpu.CompilerParams(
            vmem_limit_bytes=128 * 1024 * 1024,
        ),
    )(*args)


# speedup vs baseline: 17.0316x; 1.6069x over previous
import functools

import numpy as np
import jax
import jax.numpy as jnp
from jax.experimental import pallas as pl
from jax.experimental.pallas import tpu as pltpu

_F = 8
_DEC_LO = np.array([-0.010597401784997278, 0.032883011666982945, 0.030841381835986965, -0.18703481171888114, -0.02798376941698385, 0.6308807679295904, 0.7148465705525415, 0.23037781330885523], dtype=np.float64)
_REC_LO = _DEC_LO[::-1].copy()
_SIGN = np.array([(-1.0) ** (k + 1) for k in range(_F)], dtype=np.float64)
_DEC_HI = _REC_LO * _SIGN
_REC_HI = _DEC_HI[::-1].copy()


def _conv_valid(x, f):
    n = x.shape[1]
    m = n - _F + 1
    out = np.zeros((x.shape[0], m), dtype=x.dtype)
    for i in range(_F):
        out += f[i] * x[:, _F - 1 - i:_F - 1 - i + m]
    return out


def _conv_full(x, f):
    return _conv_valid(np.pad(x, ((0, 0), (_F - 1, _F - 1))), f)


def _dwt_np(x):
    ext = np.pad(x, ((0, 0), (_F - 1, _F - 1)), mode='symmetric')
    return _conv_valid(ext, _DEC_LO)[:, 1::2], _conv_valid(ext, _DEC_HI)[:, 1::2]


def _idwt_np(cA, cD):
    B, n = cA.shape
    ua = np.zeros((B, 2 * n), cA.dtype)
    ua[:, ::2] = cA
    ud = np.zeros((B, 2 * n), cD.dtype)
    ud[:, ::2] = cD
    full = _conv_full(ua, _REC_LO) + _conv_full(ud, _REC_HI)
    return full[:, _F - 2:2 * n]


@functools.lru_cache(maxsize=1)
def _build_stage_matrices(L=2048, max_level=8):
    ns = []
    n = L
    for _ in range(max_level):
        n = (n + _F - 1) // 2
        ns.append(n)
    casc = []
    prev = L
    for n in ns:
        eye = np.eye(prev)
        a, d = _dwt_np(eye)
        casc.append(np.concatenate([a, d], axis=1).astype(np.float32))
        prev = n
    lmat, hmat = {}, {}
    for m in set(ns):
        eye = np.eye(m)
        z = np.zeros((m, m))
        lmat[m] = _idwt_np(eye, z).astype(np.float32)
        hmat[m] = _idwt_np(z, eye).astype(np.float32)
    return tuple(ns), tuple(casc), lmat, hmat


def _tc_body(ns, B, x1_ref, *refs):
    max_level = len(ns)
    casc_refs = refs[:max_level]
    ms = sorted(set(ns))
    l_refs = dict(zip(ms, refs[max_level:max_level + len(ms)]))
    h_refs = dict(zip(ms, refs[max_level + len(ms):max_level + 2 * len(ms)]))
    (w0_ref, b0_ref, w1_ref, b1_ref, w2_ref, b2_ref,
     out_ref) = refs[max_level + 2 * len(ms):]

    def dot16(u, m_ref):
        return jnp.dot(u.astype(jnp.bfloat16), m_ref[...],
                       preferred_element_type=jnp.float32)

    def dot16_nt(u, w_ref):
        return jax.lax.dot_general(
            u.astype(jnp.bfloat16), w_ref[...].astype(jnp.bfloat16),
            (((1,), (1,)), ((), ())), preferred_element_type=jnp.float32)

    a = x1_ref[...]
    cds = []
    for layer in range(1, max_level + 1):
        n = ns[layer - 1]
        both = dot16(a, casc_refs[layer - 1])
        a = both[:, :n]
        cds.append(both[:, n:])

    def trim(r, m):
        return r[:, :m] if r.shape[1] == m + 1 else r

    T = {i: dot16(cds[i - 1], h_refs[ns[i - 1]]) for i in range(1, max_level + 1)}

    live = []
    for i in range(max_level - 1, 0, -1):
        m = ns[i - 1]
        entrant = trim(T[i + 1], m)
        stack = jnp.concatenate([trim(r, m) for r in live] + [entrant], axis=0)
        y = dot16(stack, l_refs[m])
        new_live = [y[j * B:(j + 1) * B] for j in range(len(live))]
        new_live.append(y[len(live) * B:] + T[i])
        live = new_live
    h = jnp.concatenate(live[::-1], axis=1)
    h0 = dot16_nt(jnp.maximum(h, 0.0), w0_ref) + b0_ref[...]
    h1 = dot16_nt(jnp.maximum(h0, 0.0), w1_ref) + b1_ref[...]
    out_ref[...] = dot16_nt(jnp.maximum(h1, 0.0), w2_ref) + b2_ref[...]


def kernel(x1, x2, x3, W0, b0, W1, b1, W2, b2):
    del x2, x3
    ns, casc, lmat, hmat = _build_stage_matrices()
    B = x1.shape[0]
    bf = jnp.bfloat16
    ms = sorted(set(ns))
    consts = ([jnp.asarray(c).astype(bf) for c in casc]
              + [jnp.asarray(lmat[m]).astype(bf) for m in ms]
              + [jnp.asarray(hmat[m]).astype(bf) for m in ms])
    args = ([x1] + consts
            + [W0, b0.reshape(1, -1),
               W1, b1.reshape(1, -1),
               W2, b2.reshape(1, -1)])
    body = functools.partial(_tc_body, ns, B)
    return pl.pallas_call(
        body,
        out_shape=jax.ShapeDtypeStruct((B, W2.shape[0]), jnp.float32),
        compiler_params=pltpu.CompilerParams(
            vmem_limit_bytes=128 * 1024 * 1024,
        ),
    )(*args)
